# baseline (speedup 1.0000x reference)
import functools

import jax
import jax.numpy as jnp
from jax import lax
from jax.experimental import pallas as pl
from jax.experimental.pallas import tpu as pltpu
from jax.experimental.pallas import tpu_sc as plsc

_TOPK = 0.1
_LAMBDA_LOCALITY = 0.5
_LAMBDA_ACT_L1 = 1.0

_INTERPRET = False


def _sobel_mag(a):
    B, H, W = a.shape
    p = jnp.pad(a, ((0, 0), (1, 1), (1, 1)))
    D = p[:, :, :-2] - p[:, :, 2:]
    S = p[:, :, :-2] + 2.0 * p[:, :, 1:-1] + p[:, :, 2:]
    gx = D[:, 0:H] + 2.0 * D[:, 1:H + 1] + D[:, 2:H + 2]
    gy = S[:, 0:H] - S[:, 2:H + 2]
    m2 = gx * gx + gy * gy
    return jnp.sqrt(m2), m2


def _mag_kernel(x_ref, m2b_ref, rs_ref, cs_ref):
    a = x_ref[...]
    ab = a.astype(jnp.bfloat16).astype(jnp.float32)
    mag, m2 = _sobel_mag(ab)
    rs_ref[...] = jnp.sum(mag, axis=2)
    cs_ref[...] = jnp.sum(mag, axis=1)
    m2b_ref[...] = jax.lax.bitcast_convert_type(m2, jnp.int32)


def _mask_kernel(x_ref, m2b_ref, t_ref, out_ref):
    t = t_ref[...][:, :1]
    out_ref[...] = jnp.where(m2b_ref[...] >= t[:, :, None], x_ref[...], 0.0)


def _reg_kernel(total_elems, rs_ref, cs_ref, out_ref):
    rs = rs_ref[...]
    cs = cs_ref[...]
    s = jnp.sum(rs, axis=2)

    def ent(prob):
        p = prob / s[:, :, None]
        logp = jnp.log(jnp.clip(p, 1e-38, None))
        return -jnp.sum(p * logp, axis=2)

    ex = ent(rs)
    ey = ent(cs)
    tot = jnp.sum(s, axis=1, keepdims=True)
    w = s / tot
    reg = (jnp.sum(s) / total_elems * _LAMBDA_ACT_L1
           + (jnp.mean(ex * w) + jnp.mean(ey * w)) * _LAMBDA_LOCALITY)
    out_ref[...] = reg.reshape(1, 1)


def _make_sc_thresh(nc, npix, k_top):
    mesh = plsc.VectorSubcoreMesh(core_axis_name="c", subcore_axis_name="s")
    nw = 32
    ppw = nc // nw
    nvec = npix // 16

    @functools.partial(
        pl.kernel, mesh=mesh,
        out_type=jax.ShapeDtypeStruct((nc, 16), jnp.int32),
        scratch_types=[
            pltpu.VMEM((npix,), jnp.int32),
            pltpu.VMEM((16,), jnp.int32),
            pltpu.VMEM((16,), jnp.int32),
            pltpu.SemaphoreType.DMA,
        ],
    )
    def sc_thresh(bits_hbm, out_hbm, buf, tvec, acc_ref, sem):
        wid = lax.axis_index("s") * 2 + lax.axis_index("c")

        def plane_body(j, carry):
            p = wid * ppw + j
            pltpu.async_copy(bits_hbm.at[p], buf, sem).wait()

            def body(_, c):
                lo, hi = c
                mid = lo + (hi - lo) // 2
                acc_ref[...] = jnp.zeros((16,), jnp.int32)

                def cbody(i, carry):
                    v = buf[pl.ds(i * 16, 16)]
                    acc_ref[...] = (acc_ref[...]
                                    + jnp.where(v >= mid, 1, 0).astype(jnp.int32))
                    return carry

                lax.fori_loop(0, nvec, cbody, 0)
                accv = acc_ref[...]
                cnt = accv[0]
                for i in range(1, 16):
                    cnt = cnt + accv[i]
                ge = cnt >= k_top
                return (jnp.where(ge, mid, lo), jnp.where(ge, hi, mid))

            lo, hi = lax.fori_loop(
                0, 31, body, (jnp.int32(0), jnp.int32(0x7F800001)))
            tvec[...] = jnp.full((16,), jnp.int32(0), jnp.int32) + lo
            pltpu.sync_copy(tvec, out_hbm.at[p])
            return carry

        lax.fori_loop(0, ppw, plane_body, 0)

    return sc_thresh


def kernel(x, tau):
    n, c, h, w = x.shape
    nc = n * c
    npix = h * w
    k_top = max(int(_TOPK * npix), 1)
    xr = x.reshape(nc, h, w)

    B = 32
    grid = (nc // B,)
    m2b, rs, cs = pl.pallas_call(
        _mag_kernel,
        grid=grid,
        in_specs=[pl.BlockSpec((B, h, w), lambda i: (i, 0, 0))],
        out_specs=[
            pl.BlockSpec((B, h, w), lambda i: (i, 0, 0)),
            pl.BlockSpec((B, h), lambda i: (i, 0)),
            pl.BlockSpec((B, w), lambda i: (i, 0)),
        ],
        out_shape=[
            jax.ShapeDtypeStruct((nc, h, w), jnp.int32),
            jax.ShapeDtypeStruct((nc, h), jnp.float32),
            jax.ShapeDtypeStruct((nc, w), jnp.float32),
        ],
        interpret=_INTERPRET,
    )(xr)

    thresh = _make_sc_thresh(nc, npix, k_top)(m2b.reshape(nc, npix))

    sparse = pl.pallas_call(
        _mask_kernel,
        grid=grid,
        in_specs=[
            pl.BlockSpec((B, h, w), lambda i: (i, 0, 0)),
            pl.BlockSpec((B, h, w), lambda i: (i, 0, 0)),
            pl.BlockSpec((B, 16), lambda i: (i, 0)),
        ],
        out_specs=pl.BlockSpec((B, h, w), lambda i: (i, 0, 0)),
        out_shape=jax.ShapeDtypeStruct((nc, h, w), x.dtype),
        interpret=_INTERPRET,
    )(xr, m2b, thresh)

    reg2d = pl.pallas_call(
        functools.partial(_reg_kernel, float(nc * npix)),
        out_shape=jax.ShapeDtypeStruct((1, 1), jnp.float32),
        interpret=_INTERPRET,
    )(rs.reshape(n, c, h), cs.reshape(n, c, w))

    return sparse.reshape(n, c, h, w), reg2d[0, 0]


# speedup vs baseline: 4.6155x; 4.6155x over previous
import functools

import jax
import jax.numpy as jnp
from jax import lax
from jax.experimental import pallas as pl
from jax.experimental.pallas import tpu as pltpu
from jax.experimental.pallas import tpu_sc as plsc

_TOPK = 0.1
_LAMBDA_LOCALITY = 0.5
_LAMBDA_ACT_L1 = 1.0

_INTERPRET = False


def _sobel_mag(a):
    B, H, W = a.shape
    p = jnp.pad(a, ((0, 0), (1, 1), (1, 1)))
    D = p[:, :, :-2] - p[:, :, 2:]
    S = p[:, :, :-2] + 2.0 * p[:, :, 1:-1] + p[:, :, 2:]
    gx = D[:, 0:H] + 2.0 * D[:, 1:H + 1] + D[:, 2:H + 2]
    gy = S[:, 0:H] - S[:, 2:H + 2]
    m2 = gx * gx + gy * gy
    return jnp.sqrt(m2), m2


def _mag_kernel(x_ref, m2b_ref, rs_ref, cs_ref):
    a = x_ref[...]
    ab = a.astype(jnp.bfloat16).astype(jnp.float32)
    mag, m2 = _sobel_mag(ab)
    rs_ref[...] = jnp.sum(mag, axis=2)
    cs_ref[...] = jnp.sum(mag, axis=1)
    m2b_ref[...] = jax.lax.bitcast_convert_type(m2, jnp.int32)


def _mask_kernel(x_ref, m2b_ref, t_ref, out_ref):
    t = t_ref[...][:, :1]
    out_ref[...] = jnp.where(m2b_ref[...] >= t[:, :, None], x_ref[...], 0.0)


def _reg_kernel(total_elems, rs_ref, cs_ref, out_ref):
    rs = rs_ref[...]
    cs = cs_ref[...]
    s = jnp.sum(rs, axis=2)

    def ent(prob):
        p = prob / s[:, :, None]
        logp = jnp.log(jnp.clip(p, 1e-38, None))
        return -jnp.sum(p * logp, axis=2)

    ex = ent(rs)
    ey = ent(cs)
    tot = jnp.sum(s, axis=1, keepdims=True)
    w = s / tot
    reg = (jnp.sum(s) / total_elems * _LAMBDA_ACT_L1
           + (jnp.mean(ex * w) + jnp.mean(ey * w)) * _LAMBDA_LOCALITY)
    out_ref[...] = reg.reshape(1, 1)


def _make_sc_thresh(nc, npix, k_top):
    mesh = plsc.VectorSubcoreMesh(core_axis_name="c", subcore_axis_name="s")
    nw = 32
    ppw = nc // nw
    nvec = npix // 16

    @functools.partial(
        pl.kernel, mesh=mesh,
        out_type=jax.ShapeDtypeStruct((nc, 16), jnp.int32),
        scratch_types=[
            pltpu.VMEM((npix,), jnp.int32),
            pltpu.VMEM((16,), jnp.int32),
            pltpu.VMEM((16,), jnp.int32),
            pltpu.SemaphoreType.DMA,
        ],
    )
    def sc_thresh(bits_hbm, out_hbm, buf, tvec, acc_ref, sem):
        wid = lax.axis_index("s") * 2 + lax.axis_index("c")

        def plane_body(j, carry):
            p = wid * ppw + j
            pltpu.async_copy(bits_hbm.at[p], buf, sem).wait()

            def body(_, c):
                lo, hi = c
                mid = lo + (hi - lo) // 2
                zero = jnp.zeros((16,), jnp.int32)

                def cbody(i, accs):
                    a0, a1, a2, a3 = accs
                    base = i * 64
                    w0 = jnp.where(buf[pl.ds(base, 16)] >= mid, 1, 0)
                    w1 = jnp.where(buf[pl.ds(base + 16, 16)] >= mid, 1, 0)
                    w2 = jnp.where(buf[pl.ds(base + 32, 16)] >= mid, 1, 0)
                    w3 = jnp.where(buf[pl.ds(base + 48, 16)] >= mid, 1, 0)
                    return (a0 + w0, a1 + w1, a2 + w2, a3 + w3)

                a0, a1, a2, a3 = lax.fori_loop(
                    0, nvec // 4, cbody, (zero, zero, zero, zero))
                accv = (a0 + a1) + (a2 + a3)
                cnt = accv[0]
                for i in range(1, 16):
                    cnt = cnt + accv[i]
                ge = cnt >= k_top
                return (jnp.where(ge, mid, lo), jnp.where(ge, hi, mid))

            lo, hi = lax.fori_loop(
                0, 31, body, (jnp.int32(0), jnp.int32(0x7F800001)))
            tvec[...] = jnp.full((16,), jnp.int32(0), jnp.int32) + lo
            pltpu.sync_copy(tvec, out_hbm.at[p])
            return carry

        lax.fori_loop(0, ppw, plane_body, 0)

    return sc_thresh


def kernel(x, tau):
    n, c, h, w = x.shape
    nc = n * c
    npix = h * w
    k_top = max(int(_TOPK * npix), 1)
    xr = x.reshape(nc, h, w)

    B = 32
    grid = (nc // B,)
    m2b, rs, cs = pl.pallas_call(
        _mag_kernel,
        grid=grid,
        in_specs=[pl.BlockSpec((B, h, w), lambda i: (i, 0, 0))],
        out_specs=[
            pl.BlockSpec((B, h, w), lambda i: (i, 0, 0)),
            pl.BlockSpec((B, h), lambda i: (i, 0)),
            pl.BlockSpec((B, w), lambda i: (i, 0)),
        ],
        out_shape=[
            jax.ShapeDtypeStruct((nc, h, w), jnp.int32),
            jax.ShapeDtypeStruct((nc, h), jnp.float32),
            jax.ShapeDtypeStruct((nc, w), jnp.float32),
        ],
        interpret=_INTERPRET,
    )(xr)

    thresh = _make_sc_thresh(nc, npix, k_top)(m2b.reshape(nc, npix))

    sparse = pl.pallas_call(
        _mask_kernel,
        grid=grid,
        in_specs=[
            pl.BlockSpec((B, h, w), lambda i: (i, 0, 0)),
            pl.BlockSpec((B, h, w), lambda i: (i, 0, 0)),
            pl.BlockSpec((B, 16), lambda i: (i, 0)),
        ],
        out_specs=pl.BlockSpec((B, h, w), lambda i: (i, 0, 0)),
        out_shape=jax.ShapeDtypeStruct((nc, h, w), x.dtype),
        interpret=_INTERPRET,
    )(xr, m2b, thresh)

    reg2d = pl.pallas_call(
        functools.partial(_reg_kernel, float(nc * npix)),
        out_shape=jax.ShapeDtypeStruct((1, 1), jnp.float32),
        interpret=_INTERPRET,
    )(rs.reshape(n, c, h), cs.reshape(n, c, w))

    return sparse.reshape(n, c, h, w), reg2d[0, 0]


# speedup vs baseline: 4.6414x; 1.0056x over previous
import functools

import jax
import jax.numpy as jnp
from jax import lax
from jax.experimental import pallas as pl
from jax.experimental.pallas import tpu as pltpu
from jax.experimental.pallas import tpu_sc as plsc

_TOPK = 0.1
_LAMBDA_LOCALITY = 0.5
_LAMBDA_ACT_L1 = 1.0

_INTERPRET = False


def _sobel_mag(a):
    B, H, W = a.shape
    p = jnp.pad(a, ((0, 0), (1, 1), (1, 1)))
    D = p[:, :, :-2] - p[:, :, 2:]
    S = p[:, :, :-2] + 2.0 * p[:, :, 1:-1] + p[:, :, 2:]
    gx = D[:, 0:H] + 2.0 * D[:, 1:H + 1] + D[:, 2:H + 2]
    gy = S[:, 0:H] - S[:, 2:H + 2]
    m2 = gx * gx + gy * gy
    return jnp.sqrt(m2), m2


def _mag_kernel(x_ref, m2b_ref, rs_ref, cs_ref):
    a = x_ref[...]
    ab = a.astype(jnp.bfloat16).astype(jnp.float32)
    mag, m2 = _sobel_mag(ab)
    rs_ref[...] = jnp.sum(mag, axis=2)
    cs_ref[...] = jnp.sum(mag, axis=1)
    m2b_ref[...] = jax.lax.bitcast_convert_type(m2, jnp.int32)


def _mask_kernel(x_ref, m2b_ref, t_ref, out_ref):
    t = t_ref[...][:, :1]
    out_ref[...] = jnp.where(m2b_ref[...] >= t[:, :, None], x_ref[...], 0.0)


def _reg_kernel(total_elems, rs_ref, cs_ref, out_ref):
    rs = rs_ref[...]
    cs = cs_ref[...]
    s = jnp.sum(rs, axis=2)

    def ent(prob):
        p = prob / s[:, :, None]
        logp = jnp.log(jnp.clip(p, 1e-38, None))
        return -jnp.sum(p * logp, axis=2)

    ex = ent(rs)
    ey = ent(cs)
    tot = jnp.sum(s, axis=1, keepdims=True)
    w = s / tot
    reg = (jnp.sum(s) / total_elems * _LAMBDA_ACT_L1
           + (jnp.mean(ex * w) + jnp.mean(ey * w)) * _LAMBDA_LOCALITY)
    out_ref[...] = reg.reshape(1, 1)


def _make_sc_thresh(nc, npix, k_top):
    mesh = plsc.VectorSubcoreMesh(core_axis_name="c", subcore_axis_name="s")
    nw = 32
    ppw = nc // nw
    nvec = npix // 16

    @functools.partial(
        pl.kernel, mesh=mesh,
        out_type=jax.ShapeDtypeStruct((nc, 16), jnp.int32),
        scratch_types=[
            pltpu.VMEM((2, npix), jnp.int32),
            pltpu.VMEM((16,), jnp.int32),
            pltpu.SemaphoreType.DMA((2,)),
        ],
    )
    def sc_thresh(bits_hbm, out_hbm, buf, tvec, sem):
        wid = lax.axis_index("s") * 2 + lax.axis_index("c")
        base_p = wid * ppw

        def dma(j, slot):
            return pltpu.make_async_copy(
                bits_hbm.at[base_p + j], buf.at[slot], sem.at[slot])

        dma(0, 0).start()

        def plane_body(j, carry):
            slot = lax.rem(j, 2)
            dma(j, slot).wait()
            nxt = jnp.minimum(j + 1, ppw - 1)
            dma(nxt, lax.rem(j + 1, 2)).start()

            def body(_, c):
                lo, hi = c
                mid = lo + (hi - lo) // 2
                zero = jnp.zeros((16,), jnp.int32)

                def cbody(i, accs):
                    a0, a1, a2, a3 = accs
                    base = i * 64
                    w0 = jnp.where(buf[slot, pl.ds(base, 16)] >= mid, 1, 0)
                    w1 = jnp.where(buf[slot, pl.ds(base + 16, 16)] >= mid, 1, 0)
                    w2 = jnp.where(buf[slot, pl.ds(base + 32, 16)] >= mid, 1, 0)
                    w3 = jnp.where(buf[slot, pl.ds(base + 48, 16)] >= mid, 1, 0)
                    return (a0 + w0, a1 + w1, a2 + w2, a3 + w3)

                a0, a1, a2, a3 = lax.fori_loop(
                    0, nvec // 4, cbody, (zero, zero, zero, zero))
                accv = (a0 + a1) + (a2 + a3)
                cnt = accv[0]
                for i in range(1, 16):
                    cnt = cnt + accv[i]
                ge = cnt >= k_top
                return (jnp.where(ge, mid, lo), jnp.where(ge, hi, mid))

            lo, hi = lax.fori_loop(
                0, 31, body, (jnp.int32(0), jnp.int32(0x7F800001)))
            tvec[...] = jnp.full((16,), jnp.int32(0), jnp.int32) + lo
            pltpu.sync_copy(tvec, out_hbm.at[base_p + j])
            return carry

        lax.fori_loop(0, ppw, plane_body, 0)
        dma(ppw - 1, ppw % 2).wait()

    return sc_thresh


def kernel(x, tau):
    n, c, h, w = x.shape
    nc = n * c
    npix = h * w
    k_top = max(int(_TOPK * npix), 1)
    xr = x.reshape(nc, h, w)

    B = 32
    grid = (nc // B,)
    m2b, rs, cs = pl.pallas_call(
        _mag_kernel,
        grid=grid,
        in_specs=[pl.BlockSpec((B, h, w), lambda i: (i, 0, 0))],
        out_specs=[
            pl.BlockSpec((B, h, w), lambda i: (i, 0, 0)),
            pl.BlockSpec((B, h), lambda i: (i, 0)),
            pl.BlockSpec((B, w), lambda i: (i, 0)),
        ],
        out_shape=[
            jax.ShapeDtypeStruct((nc, h, w), jnp.int32),
            jax.ShapeDtypeStruct((nc, h), jnp.float32),
            jax.ShapeDtypeStruct((nc, w), jnp.float32),
        ],
        interpret=_INTERPRET,
    )(xr)

    thresh = _make_sc_thresh(nc, npix, k_top)(m2b.reshape(nc, npix))

    sparse = pl.pallas_call(
        _mask_kernel,
        grid=grid,
        in_specs=[
            pl.BlockSpec((B, h, w), lambda i: (i, 0, 0)),
            pl.BlockSpec((B, h, w), lambda i: (i, 0, 0)),
            pl.BlockSpec((B, 16), lambda i: (i, 0)),
        ],
        out_specs=pl.BlockSpec((B, h, w), lambda i: (i, 0, 0)),
        out_shape=jax.ShapeDtypeStruct((nc, h, w), x.dtype),
        interpret=_INTERPRET,
    )(xr, m2b, thresh)

    reg2d = pl.pallas_call(
        functools.partial(_reg_kernel, float(nc * npix)),
        out_shape=jax.ShapeDtypeStruct((1, 1), jnp.float32),
        interpret=_INTERPRET,
    )(rs.reshape(n, c, h), cs.reshape(n, c, w))

    return sparse.reshape(n, c, h, w), reg2d[0, 0]


# speedup vs baseline: 6.1438x; 1.3237x over previous
import functools

import jax
import jax.numpy as jnp
from jax import lax
from jax.experimental import pallas as pl
from jax.experimental.pallas import tpu as pltpu
from jax.experimental.pallas import tpu_sc as plsc

_TOPK = 0.1
_LAMBDA_LOCALITY = 0.5
_LAMBDA_ACT_L1 = 1.0


def _sobel_mag(a):
    B, H, W = a.shape
    p = jnp.pad(a, ((0, 0), (1, 1), (1, 1)))
    D = p[:, :, :-2] - p[:, :, 2:]
    S = p[:, :, :-2] + 2.0 * p[:, :, 1:-1] + p[:, :, 2:]
    gx = D[:, 0:H] + 2.0 * D[:, 1:H + 1] + D[:, 2:H + 2]
    gy = S[:, 0:H] - S[:, 2:H + 2]
    m2 = gx * gx + gy * gy
    return jnp.sqrt(m2), m2


def _mag_kernel(x_ref, m2b_ref, rs_ref, cs_ref):
    a = x_ref[...]
    ab = a.astype(jnp.bfloat16).astype(jnp.float32)
    mag, m2 = _sobel_mag(ab)
    rs_ref[...] = jnp.sum(mag, axis=2)
    cs_ref[...] = jnp.sum(mag, axis=1)
    m2b_ref[...] = jax.lax.bitcast_convert_type(m2, jnp.int32)


def _mask_kernel(x_ref, m2b_ref, t_ref, out_ref):
    t = t_ref[...][:, :1]
    out_ref[...] = jnp.where(m2b_ref[...] >= t[:, :, None], x_ref[...], 0.0)


def _reg_kernel(total_elems, rs_ref, cs_ref, out_ref):
    rs = rs_ref[...]
    cs = cs_ref[...]
    s = jnp.sum(rs, axis=2)

    def ent(prob):
        p = prob / s[:, :, None]
        logp = jnp.log(jnp.clip(p, 1e-38, None))
        return -jnp.sum(p * logp, axis=2)

    ex = ent(rs)
    ey = ent(cs)
    tot = jnp.sum(s, axis=1, keepdims=True)
    w = s / tot
    reg = (jnp.sum(s) / total_elems * _LAMBDA_ACT_L1
           + (jnp.mean(ex * w) + jnp.mean(ey * w)) * _LAMBDA_LOCALITY)
    out_ref[...] = reg.reshape(1, 1)


def _make_sc_thresh(nc, npix, k_top):
    mesh = plsc.VectorSubcoreMesh(core_axis_name="c", subcore_axis_name="s")
    nw = 32
    ppw = nc // nw
    nvec = npix // 16

    @functools.partial(
        pl.kernel, mesh=mesh,
        out_type=jax.ShapeDtypeStruct((nc, 16), jnp.int32),
        scratch_types=[
            pltpu.VMEM((2, npix), jnp.int32),
            pltpu.VMEM((16,), jnp.int32),
            pltpu.SemaphoreType.DMA((2,)),
        ],
    )
    def sc_thresh(bits_hbm, out_hbm, buf, tvec, sem):
        wid = lax.axis_index("s") * 2 + lax.axis_index("c")
        base_p = wid * ppw

        def dma(j, slot):
            return pltpu.make_async_copy(
                bits_hbm.at[base_p + j], buf.at[slot], sem.at[slot])

        dma(0, 0).start()

        def plane_body(j, carry):
            slot = lax.rem(j, 2)
            dma(j, slot).wait()
            nxt = jnp.minimum(j + 1, ppw - 1)
            dma(nxt, lax.rem(j + 1, 2)).start()

            def body(_, c):
                lo, hi = c
                mid = lo + (hi - lo) // 2
                zero = jnp.zeros((16,), jnp.int32)

                def cbody(i, accs):
                    base = i * 128
                    return tuple(
                        a + jnp.where(
                            buf[slot, pl.ds(base + 16 * u, 16)] >= mid, 1, 0)
                        for u, a in enumerate(accs))

                accs = lax.fori_loop(
                    0, nvec // 8, cbody, (zero,) * 8)
                accv = (((accs[0] + accs[1]) + (accs[2] + accs[3]))
                        + ((accs[4] + accs[5]) + (accs[6] + accs[7])))
                cnt = accv[0]
                for i in range(1, 16):
                    cnt = cnt + accv[i]
                ge = cnt >= k_top
                return (jnp.where(ge, mid, lo), jnp.where(ge, hi, mid))

            lo, hi = lax.fori_loop(
                0, 31, body, (jnp.int32(0), jnp.int32(0x7F800001)))
            tvec[...] = jnp.full((16,), jnp.int32(0), jnp.int32) + lo
            pltpu.sync_copy(tvec, out_hbm.at[base_p + j])
            return carry

        lax.fori_loop(0, ppw, plane_body, 0)
        dma(ppw - 1, ppw % 2).wait()

    return sc_thresh


def kernel(x, tau):
    n, c, h, w = x.shape
    nc = n * c
    npix = h * w
    k_top = max(int(_TOPK * npix), 1)
    xr = x.reshape(nc, h, w)

    B = 32
    grid = (nc // B,)
    m2b, rs, cs = pl.pallas_call(
        _mag_kernel,
        grid=grid,
        in_specs=[pl.BlockSpec((B, h, w), lambda i: (i, 0, 0))],
        out_specs=[
            pl.BlockSpec((B, h, w), lambda i: (i, 0, 0)),
            pl.BlockSpec((B, h), lambda i: (i, 0)),
            pl.BlockSpec((B, w), lambda i: (i, 0)),
        ],
        out_shape=[
            jax.ShapeDtypeStruct((nc, h, w), jnp.int32),
            jax.ShapeDtypeStruct((nc, h), jnp.float32),
            jax.ShapeDtypeStruct((nc, w), jnp.float32),
        ],
    )(xr)

    thresh = _make_sc_thresh(nc, npix, k_top)(m2b.reshape(nc, npix))

    sparse = pl.pallas_call(
        _mask_kernel,
        grid=grid,
        in_specs=[
            pl.BlockSpec((B, h, w), lambda i: (i, 0, 0)),
            pl.BlockSpec((B, h, w), lambda i: (i, 0, 0)),
            pl.BlockSpec((B, 16), lambda i: (i, 0)),
        ],
        out_specs=pl.BlockSpec((B, h, w), lambda i: (i, 0, 0)),
        out_shape=jax.ShapeDtypeStruct((nc, h, w), x.dtype),
    )(xr, m2b, thresh)

    reg2d = pl.pallas_call(
        functools.partial(_reg_kernel, float(nc * npix)),
        out_shape=jax.ShapeDtypeStruct((1, 1), jnp.float32),
    )(rs.reshape(n, c, h), cs.reshape(n, c, w))

    return sparse.reshape(n, c, h, w), reg2d[0, 0]
